# baseline (speedup 1.0000x reference)
import functools

import jax
import jax.numpy as jnp
from jax import lax
from jax.experimental import pallas as pl
from jax.experimental.pallas import tpu as pltpu
from jax.experimental.pallas import tpu_sc as plsc

B, N, K, D, T = 4096, 50000, 32, 128, 2

_NC, _NS = 2, 16
_NW = _NC * _NS

_DN = (((1,), (1,)), ((), ()))

_TBLK = 2000


def _fuse_table_body(e, p, wf, bf, fo):
    fo[...] = jnp.maximum(
        lax.dot_general(e[...], wf[:, :D], _DN,
                        preferred_element_type=jnp.float32)
        + lax.dot_general(p[...], wf[:, D:], _DN,
                          preferred_element_type=jnp.float32)
        + bf[...], 0.0)


def _fuse_table(e, p, wf, bf2):
    tab_spec = pl.BlockSpec((_TBLK, D), lambda i: (i, 0))
    return pl.pallas_call(
        _fuse_table_body,
        grid=(N // _TBLK,),
        in_specs=[tab_spec, tab_spec,
                  pl.BlockSpec((D, 2 * D), lambda i: (0, 0)),
                  pl.BlockSpec((1, D), lambda i: (0, 0))],
        out_specs=tab_spec,
        out_shape=jax.ShapeDtypeStruct((N, D), jnp.float32),
    )(e, p, wf, bf2)


_PW = (K * B) // _NW
_C = 256
_NCHUNK = _PW // _C
_PWN = B // _NW


def _ring_gather(wid, tab, idxs, out, bufs, gsem, wsem,
                 pw=_PW, nchunk=_NCHUNK):
    def pair_body(jj, carry):
        for p in range(2):
            j = jj * 2 + p
            base = wid * pw + j * _C
            idxv, rowsv = bufs[p]

            @pl.when(jj > 0)
            def _drain():
                pltpu.make_async_copy(
                    rowsv, out.at[pl.ds(base - 2 * _C, _C)], wsem).wait()

            pltpu.sync_copy(idxs.at[pl.ds(base, _C)], idxv)
            pltpu.async_copy(tab.at[idxv], rowsv, gsem).wait()
            pltpu.async_copy(rowsv, out.at[pl.ds(base, _C)], wsem)
        return carry
    lax.fori_loop(0, nchunk // 2, pair_body, 0)
    for p in range(2):
        base = wid * pw + (nchunk - 2 + p) * _C
        pltpu.make_async_copy(bufs[p][1], out.at[pl.ds(base, _C)],
                              wsem).wait()


def _gather_t0_body(f0, idx0, nemb, nprof, nds, out0, one, onp,
                    idxv0, idxv1, rowsv0, rowsv1, idxn, rowsn, gsem, wsem):
    wid = lax.axis_index("s") * _NC + lax.axis_index("c")
    _ring_gather(wid, f0, idx0, out0,
                 ((idxv0, rowsv0), (idxv1, rowsv1)), gsem, wsem)
    nb = wid * _PWN
    pltpu.sync_copy(nds.at[pl.ds(nb, _PWN)], idxn)
    pltpu.async_copy(nemb.at[idxn], rowsn, gsem).wait()
    pltpu.sync_copy(rowsn, one.at[pl.ds(nb, _PWN)])
    pltpu.async_copy(nprof.at[idxn], rowsn, gsem).wait()
    pltpu.sync_copy(rowsn, onp.at[pl.ds(nb, _PWN)])


def _make_gather_t1_body(pw, nchunk):
    def _gather_t1_body(f1, idx1, out1,
                        idxv0, idxv1, rowsv0, rowsv1, gsem, wsem):
        wid = lax.axis_index("s") * _NC + lax.axis_index("c")
        _ring_gather(wid, f1, idx1, out1,
                     ((idxv0, rowsv0), (idxv1, rowsv1)), gsem, wsem,
                     pw=pw, nchunk=nchunk)
    return _gather_t1_body


_RING_SCRATCH = [
    pltpu.VMEM((_C,), jnp.int32),
    pltpu.VMEM((_C,), jnp.int32),
    pltpu.VMEM((_C, D), jnp.float32),
    pltpu.VMEM((_C, D), jnp.float32),
]


@functools.cache
def _build_gather_t0():
    return functools.partial(
        pl.kernel,
        out_type=[
            jax.ShapeDtypeStruct((K * B, D), jnp.float32),
            jax.ShapeDtypeStruct((B, D), jnp.float32),
            jax.ShapeDtypeStruct((B, D), jnp.float32),
        ],
        mesh=plsc.VectorSubcoreMesh(core_axis_name="c", subcore_axis_name="s"),
        scratch_types=_RING_SCRATCH + [
            pltpu.VMEM((_PWN,), jnp.int32),
            pltpu.VMEM((_PWN, D), jnp.float32),
            pltpu.SemaphoreType.DMA,
            pltpu.SemaphoreType.DMA,
        ],
    )(_gather_t0_body)


@functools.cache
def _build_gather_t1(hb):
    pw = (K * hb) // _NW
    return functools.partial(
        pl.kernel,
        out_type=jax.ShapeDtypeStruct((K * hb, D), jnp.float32),
        mesh=plsc.VectorSubcoreMesh(core_axis_name="c", subcore_axis_name="s"),
        scratch_types=_RING_SCRATCH + [
            pltpu.SemaphoreType.DMA,
            pltpu.SemaphoreType.DMA,
        ],
    )(_make_gather_t1_body(pw, pw // _C))


_BB = 512


def _attention(q, nf_ref, w1v, b1v):
    ones_dk = jnp.ones((D, K), jnp.float32)
    kiota = lax.broadcasted_iota(jnp.int32, (1, K), 1)
    ones_1d = jnp.ones((1, D), jnp.float32)
    dn_nt = (((1,), (0,)), ((), ()))
    s = jnp.zeros((_BB, K), jnp.float32)
    for k in range(K):
        s = s + lax.dot_general(
            q * nf_ref[k], ones_dk * (kiota == k).astype(jnp.float32),
            dn_nt, preferred_element_type=jnp.float32)
    m = jnp.max(s, axis=1, keepdims=True)
    e = jnp.exp(s - m)
    att_k = e / jnp.sum(e, axis=1, keepdims=True)
    feat = jnp.zeros((_BB, D), jnp.float32)
    for k in range(K):
        ab = lax.dot_general(att_k[:, k:k + 1], ones_1d, dn_nt,
                             preferred_element_type=jnp.float32)
        feat = feat + ab * nf_ref[k]
    return jnp.maximum(
        lax.dot_general(feat, w1v, _DN, preferred_element_type=jnp.float32)
        + b1v, 0.0)


def _attend_t0_body(ne, npf, nf0, wf, bf, w1, b1, qo, agg0o):
    q = jnp.maximum(
        lax.dot_general(ne[...], wf[:, :D], _DN,
                        preferred_element_type=jnp.float32)
        + lax.dot_general(npf[...], wf[:, D:], _DN,
                          preferred_element_type=jnp.float32)
        + bf[...], 0.0)
    qo[...] = q
    agg0o[...] = _attention(q, nf0, w1[...], b1[...])


def _attend_t0(ne, npf, nf0, wf, bf2, w1, b12):
    row_spec = pl.BlockSpec((_BB, D), lambda i: (i, 0))
    nf_spec = pl.BlockSpec((K, _BB, D), lambda i: (0, i, 0))
    full = lambda shape: pl.BlockSpec(shape, lambda i: tuple(0 for _ in shape))
    return pl.pallas_call(
        _attend_t0_body,
        grid=(B // _BB,),
        in_specs=[row_spec, row_spec, nf_spec,
                  full((D, 2 * D)), full((1, D)),
                  full((D, D)), full((1, D))],
        out_specs=[row_spec, row_spec],
        out_shape=[
            jax.ShapeDtypeStruct((B, D), jnp.float32),
            jax.ShapeDtypeStruct((B, D), jnp.float32),
        ],
    )(ne, npf, nf0, wf, bf2, w1, b12)


def _attend_t1_body(qr, agg0r, nf1, w1, b1, w2, b2, w, bb, wt, combo, atto):
    q = qr[...]
    agg0 = agg0r[...]
    agg1 = _attention(q, nf1, w1[...], b1[...])
    ta = jnp.concatenate([agg0, agg1], axis=1)
    mta = lax.dot_general(ta, wt[...], _DN, preferred_element_type=jnp.float32)
    mm = jnp.max(mta, axis=1, keepdims=True)
    ee = jnp.exp(mta - mm)
    att = ee / jnp.sum(ee, axis=1, keepdims=True)
    fin = att[:, 0:1] * agg0 + att[:, 1:2] * agg1
    fin = jnp.maximum(
        lax.dot_general(fin, w2[...], _DN, preferred_element_type=jnp.float32)
        + b2[...], 0.0)
    comb = jnp.concatenate([q, fin], axis=1)
    combo[...] = jnp.maximum(
        lax.dot_general(comb, w[...], _DN, preferred_element_type=jnp.float32)
        + bb[...], 0.0)
    atto[...] = att


def _attend_t1(qn, agg0, nf1, w1, b12, w2, b22, w, b2d, wt):
    hb = qn.shape[0]
    row_spec = pl.BlockSpec((_BB, D), lambda i: (i, 0))
    nf_spec = pl.BlockSpec((K, _BB, D), lambda i: (0, i, 0))
    full = lambda shape: pl.BlockSpec(shape, lambda i: tuple(0 for _ in shape))
    return pl.pallas_call(
        _attend_t1_body,
        grid=(hb // _BB,),
        in_specs=[row_spec, row_spec, nf_spec,
                  full((D, D)), full((1, D)),
                  full((D, D)), full((1, D)),
                  full((D, 2 * D)), full((1, D)),
                  full((T, 2 * D))],
        out_specs=[row_spec, pl.BlockSpec((_BB, T), lambda i: (i, 0))],
        out_shape=[
            jax.ShapeDtypeStruct((hb, D), jnp.float32),
            jax.ShapeDtypeStruct((hb, T), jnp.float32),
        ],
    )(qn, agg0, nf1, w1, b12, w2, b22, w, b2d, wt)


def kernel(nodes, neigh_idx_0, neigh_idx_1, node_emb, node_prof,
           neigh_emb_0, neigh_prof_0, neigh_emb_1, neigh_prof_1,
           Wf, bf, W1, b1, W2, b2, W, b, Wt):
    nodes_i = nodes.astype(jnp.int32)
    idx0t = neigh_idx_0.astype(jnp.int32).T.reshape(-1)
    bf2 = bf.reshape(1, D)

    f0 = _fuse_table(neigh_emb_0, neigh_prof_0, Wf, bf2)
    nf0, ne, npf = _build_gather_t0()(f0, idx0t, node_emb, node_prof,
                                      nodes_i)
    f1 = _fuse_table(neigh_emb_1, neigh_prof_1, Wf, bf2)
    qn, agg0 = _attend_t0(ne, npf, nf0.reshape(K, B, D), Wf, bf2,
                          W1, b1.reshape(1, D))
    hb = B // 2
    i1h = neigh_idx_1.astype(jnp.int32)
    combs, atts = [], []
    nf1h = [_build_gather_t1(hb)(f1, i1h[h * hb:(h + 1) * hb].T.reshape(-1))
            for h in range(2)]
    for h in range(2):
        sl = slice(h * hb, (h + 1) * hb)
        comb_h, att_h = _attend_t1(qn[sl], agg0[sl],
                                   nf1h[h].reshape(K, hb, D),
                                   W1, b1.reshape(1, D), W2,
                                   b2.reshape(1, D), W, b.reshape(1, D), Wt)
        combs.append(comb_h)
        atts.append(att_h)
    comb = jnp.concatenate(combs, axis=0)
    att = jnp.concatenate(atts, axis=0)
    return comb, att.reshape(B, T, 1)
